# baseline (speedup 1.0000x reference)
import functools

import jax
import jax.numpy as jnp
from jax import lax
from jax.experimental import pallas as pl
from jax.experimental.pallas import tpu as pltpu
from jax.experimental.pallas import tpu_sc as plsc

VOCAB = 1000000
EMBED_DIM = 64
BATCH = 16384
NUM_CLASS = 40

_NC = 2
_NS = 16
_NW = _NC * _NS
_B_PER_W = BATCH // _NW
_CHUNK = 128
_NCHUNK = _B_PER_W // _CHUNK


def _gather_sc(indexes, table):
    mesh = plsc.VectorSubcoreMesh(core_axis_name="c", subcore_axis_name="s")

    @functools.partial(
        pl.kernel,
        mesh=mesh,
        out_type=jax.ShapeDtypeStruct((BATCH, EMBED_DIM), jnp.float32),
        compiler_params=pltpu.CompilerParams(use_tc_tiling_on_sc=False),
        scratch_types=[
            pltpu.VMEM((_NCHUNK, _CHUNK), jnp.int32),
            pltpu.VMEM((_B_PER_W, EMBED_DIM), jnp.float32),
            pltpu.SemaphoreType.DMA,
        ],
    )
    def k(idx_hbm, table_hbm, out_hbm, idx_v, rows_v, sem):
        wid = lax.axis_index("s") * _NC + lax.axis_index("c")
        base = wid * _B_PER_W
        for j in range(_NCHUNK):
            pltpu.sync_copy(idx_hbm.at[pl.ds(base + j * _CHUNK, _CHUNK)],
                            idx_v.at[j])
        copies = [
            pltpu.async_copy(table_hbm.at[idx_v.at[j]],
                             rows_v.at[pl.ds(j * _CHUNK, _CHUNK)], sem)
            for j in range(_NCHUNK)
        ]
        for c in copies:
            c.wait()
        pltpu.sync_copy(rows_v, out_hbm.at[pl.ds(base, _B_PER_W)])

    return k(indexes, table)


def _mm_body(x_ref, wt_ref, b_ref, o_ref):
    o_ref[...] = (
        jnp.dot(x_ref[...], wt_ref[...], preferred_element_type=jnp.float32)
        + b_ref[...]
    )


def _classify_tc(x, W, b):
    blk = 4096
    wt = W.T
    b2 = b.reshape(1, NUM_CLASS)
    return pl.pallas_call(
        _mm_body,
        grid=(BATCH // blk,),
        in_specs=[
            pl.BlockSpec((blk, EMBED_DIM), lambda i: (i, 0)),
            pl.BlockSpec((EMBED_DIM, NUM_CLASS), lambda i: (0, 0)),
            pl.BlockSpec((1, NUM_CLASS), lambda i: (0, 0)),
        ],
        out_specs=pl.BlockSpec((blk, NUM_CLASS), lambda i: (i, 0)),
        out_shape=jax.ShapeDtypeStruct((BATCH, NUM_CLASS), jnp.float32),
    )(x, wt, b2)


def kernel(indexes, table, W, b):
    x = _gather_sc(indexes, table)
    return _classify_tc(x, W, b)


# speedup vs baseline: 1.6950x; 1.6950x over previous
import functools

import jax
import jax.numpy as jnp
from jax import lax
from jax.experimental import pallas as pl
from jax.experimental.pallas import tpu as pltpu
from jax.experimental.pallas import tpu_sc as plsc

VOCAB = 1000000
EMBED_DIM = 64
BATCH = 16384
NUM_CLASS = 40

_NC = 2
_NS = 16
_NW = _NC * _NS
_B_PER_W = BATCH // _NW
_CHUNK = 128
_NCHUNK = _B_PER_W // _CHUNK


_UNROLL = 8


def _gather_sc(indexes, table):
    mesh = plsc.VectorSubcoreMesh(core_axis_name="c", subcore_axis_name="s")

    @functools.partial(
        pl.kernel,
        mesh=mesh,
        out_type=jax.ShapeDtypeStruct((BATCH, EMBED_DIM), jnp.float32),
        scratch_types=[
            pltpu.VMEM((_B_PER_W,), jnp.int32),
            pltpu.VMEM((_B_PER_W, EMBED_DIM), jnp.float32),
            pltpu.SemaphoreType.DMA,
        ],
    )
    def k(idx_hbm, table_hbm, out_hbm, idx_v, rows_v, sem):
        wid = lax.axis_index("s") * _NC + lax.axis_index("c")
        base = wid * _B_PER_W
        pltpu.sync_copy(idx_hbm.at[pl.ds(base, _B_PER_W)], idx_v)

        def body(j, carry):
            base16 = j * 16
            iv = idx_v[pl.ds(base16, 16)]
            for u in range(16):
                pltpu.async_copy(
                    table_hbm.at[pl.ds(iv[u], 1)],
                    rows_v.at[pl.ds(base16 + u, 1)],
                    sem,
                )
            return carry

        lax.fori_loop(0, _B_PER_W // 16, body, 0)
        pltpu.make_async_copy(
            table_hbm.at[pl.ds(0, _B_PER_W)], rows_v, sem
        ).wait()
        pltpu.sync_copy(rows_v, out_hbm.at[pl.ds(base, _B_PER_W)])

    return k(indexes, table)


def _mm_body(x_ref, wt_ref, b_ref, o_ref):
    o_ref[...] = (
        jnp.dot(x_ref[...], wt_ref[...], preferred_element_type=jnp.float32)
        + b_ref[...]
    )


def _classify_tc(x, W, b):
    blk = 4096
    wt = W.T
    b2 = b.reshape(1, NUM_CLASS)
    return pl.pallas_call(
        _mm_body,
        grid=(BATCH // blk,),
        in_specs=[
            pl.BlockSpec((blk, EMBED_DIM), lambda i: (i, 0)),
            pl.BlockSpec((EMBED_DIM, NUM_CLASS), lambda i: (0, 0)),
            pl.BlockSpec((1, NUM_CLASS), lambda i: (0, 0)),
        ],
        out_specs=pl.BlockSpec((blk, NUM_CLASS), lambda i: (i, 0)),
        out_shape=jax.ShapeDtypeStruct((BATCH, NUM_CLASS), jnp.float32),
    )(x, wt, b2)


def kernel(indexes, table, W, b):
    x = _gather_sc(indexes, table)
    return _classify_tc(x, W, b)


# speedup vs baseline: 1.8585x; 1.0965x over previous
import functools

import jax
import jax.numpy as jnp
from jax import lax
from jax.experimental import pallas as pl
from jax.experimental.pallas import tpu as pltpu
from jax.experimental.pallas import tpu_sc as plsc

VOCAB = 1000000
EMBED_DIM = 64
BATCH = 16384
NUM_CLASS = 40

_NC = 2
_NS = 16
_NW = _NC * _NS
_B_PER_W = BATCH // _NW

_TAIL_START = (VOCAB // 128) * 128
_TAIL = VOCAB - _TAIL_START
_VT_MAX = _TAIL_START - 128


def _gather_sc(indexes, tableT):
    mesh = plsc.VectorSubcoreMesh(core_axis_name="c", subcore_axis_name="s")

    @functools.partial(
        pl.kernel,
        mesh=mesh,
        out_type=jax.ShapeDtypeStruct((BATCH, EMBED_DIM), jnp.float32),
        compiler_params=pltpu.CompilerParams(needs_layout_passes=False),
        scratch_types=[
            pltpu.VMEM((_B_PER_W,), jnp.int32),
            pltpu.VMEM((4, EMBED_DIM, 128), jnp.float32),
            pltpu.VMEM((EMBED_DIM, _TAIL), jnp.float32),
            pltpu.VMEM((_B_PER_W, EMBED_DIM), jnp.float32),
            pltpu.SemaphoreType.DMA,
        ],
    )
    def k(idx_hbm, tableT_hbm, out_hbm, idx_v, tiles_v, tail_v, rows_v, sem):
        wid = lax.axis_index("s") * _NC + lax.axis_index("c")
        base = wid * _B_PER_W
        pltpu.sync_copy(idx_hbm.at[pl.ds(base, _B_PER_W)], idx_v)
        pltpu.sync_copy(tableT_hbm.at[:, pl.ds(_TAIL_START, _TAIL)], tail_v)
        d4 = [lax.iota(jnp.int32, 16) + 16 * kk for kk in range(4)]

        def body(t, carry):
            iv = idx_v[pl.ds(t * 16, 16)]
            vt = jnp.minimum(iv & ~127, _VT_MAX)
            vin = iv & 127
            tail = (iv >= _TAIL_START).astype(jnp.int32)
            for quarter in range(4):
                cps = []
                for u in range(4):
                    lane = quarter * 4 + u
                    off = pl.multiple_of(vt[lane], 128)
                    cps.append(
                        pltpu.async_copy(
                            tableT_hbm.at[:, pl.ds(off, 128)],
                            tiles_v.at[u],
                            sem,
                        )
                    )
                for cp in cps:
                    cp.wait()
                for u in range(4):
                    lane = quarter * 4 + u
                    jj = t * 16 + lane
                    c16 = jnp.full((16,), vin[lane], jnp.int32)
                    mf16 = jnp.full((16,), tail[lane], jnp.int32).astype(
                        jnp.float32
                    )
                    j16 = jnp.full((16,), jj, jnp.int32)
                    for kk in range(4):
                        v_main = plsc.load_gather(tiles_v.at[u], [d4[kk], c16])
                        v_tail = plsc.load_gather(tail_v, [d4[kk], c16])
                        val = v_main + (v_tail - v_main) * mf16
                        plsc.store_scatter(rows_v, [j16, d4[kk]], val)
            return carry

        lax.fori_loop(0, _B_PER_W // 16, body, 0)
        pltpu.sync_copy(rows_v, out_hbm.at[pl.ds(base, _B_PER_W)])

    return k(indexes, tableT)


def _mm_body(x_ref, wt_ref, b_ref, o_ref):
    o_ref[...] = (
        jnp.dot(x_ref[...], wt_ref[...], preferred_element_type=jnp.float32)
        + b_ref[...]
    )


def _classify_tc(x, W, b):
    blk = 4096
    wt = W.T
    b2 = b.reshape(1, NUM_CLASS)
    return pl.pallas_call(
        _mm_body,
        grid=(BATCH // blk,),
        in_specs=[
            pl.BlockSpec((blk, EMBED_DIM), lambda i: (i, 0)),
            pl.BlockSpec((EMBED_DIM, NUM_CLASS), lambda i: (0, 0)),
            pl.BlockSpec((1, NUM_CLASS), lambda i: (0, 0)),
        ],
        out_specs=pl.BlockSpec((blk, NUM_CLASS), lambda i: (i, 0)),
        out_shape=jax.ShapeDtypeStruct((BATCH, NUM_CLASS), jnp.float32),
    )(x, wt, b2)


def kernel(indexes, table, W, b):
    x = _gather_sc(indexes, table.T)
    return _classify_tc(x, W, b)


# speedup vs baseline: 2.4291x; 1.3070x over previous
import functools

import jax
import jax.numpy as jnp
from jax import lax
from jax.experimental import pallas as pl
from jax.experimental.pallas import tpu as pltpu
from jax.experimental.pallas import tpu_sc as plsc

VOCAB = 1000000
EMBED_DIM = 64
BATCH = 16384
NUM_CLASS = 40

_NC = 2
_NS = 16
_NW = _NC * _NS
_B_PER_W = BATCH // _NW

_TAIL_START = (VOCAB // 128) * 128
_TAIL = VOCAB - _TAIL_START
_VT_MAX = _TAIL_START - 128


def _gather_sc(indexes, tableT):
    mesh = plsc.VectorSubcoreMesh(core_axis_name="c", subcore_axis_name="s")

    @functools.partial(
        pl.kernel,
        mesh=mesh,
        out_type=jax.ShapeDtypeStruct((BATCH, EMBED_DIM), jnp.float32),
        compiler_params=pltpu.CompilerParams(needs_layout_passes=False),
        scratch_types=[
            pltpu.VMEM((_B_PER_W,), jnp.int32),
            pltpu.VMEM((4, EMBED_DIM, 128), jnp.float32),
            pltpu.VMEM((EMBED_DIM, _TAIL), jnp.float32),
            pltpu.VMEM((_B_PER_W, EMBED_DIM), jnp.float32),
            pltpu.SemaphoreType.DMA,
            pltpu.SemaphoreType.DMA,
            pltpu.SemaphoreType.DMA,
            pltpu.SemaphoreType.DMA,
        ],
    )
    def k(idx_hbm, tableT_hbm, out_hbm, idx_v, tiles_v, tail_v, rows_v,
          s0, s1, s2, s3):
        sems = [s0, s1, s2, s3]
        wid = lax.axis_index("s") * _NC + lax.axis_index("c")
        base = wid * _B_PER_W
        pltpu.sync_copy(idx_hbm.at[pl.ds(base, _B_PER_W)], idx_v)
        pltpu.sync_copy(tableT_hbm.at[:, pl.ds(_TAIL_START, _TAIL)], tail_v)
        d4 = [lax.iota(jnp.int32, 16) + 16 * kk for kk in range(4)]

        def _fetch(off, slot):
            pltpu.async_copy(
                tableT_hbm.at[:, pl.ds(off, 128)], tiles_v.at[slot],
                sems[slot],
            )

        def _slot_wait(slot):
            pltpu.make_async_copy(
                tableT_hbm.at[:, pl.ds(0, 128)], tiles_v.at[slot], sems[slot]
            ).wait()

        iv0 = idx_v[pl.ds(0, 16)]
        vt0 = jnp.minimum(iv0 & ~127, _VT_MAX)
        for u in range(4):
            _fetch(pl.multiple_of(vt0[u], 128), u)

        def body(t, carry):
            iv = idx_v[pl.ds(t * 16, 16)]
            vt = jnp.minimum(iv & ~127, _VT_MAX)
            vin = iv & 127
            tail = (iv >= _TAIL_START).astype(jnp.int32)
            tn = jnp.minimum(t + 1, _B_PER_W // 16 - 1)
            ivn = idx_v[pl.ds(tn * 16, 16)]
            vtn = jnp.minimum(ivn & ~127, _VT_MAX)
            for lane in range(16):
                slot = lane % 4
                _slot_wait(slot)
                jj = t * 16 + lane
                c16 = jnp.full((16,), vin[lane], jnp.int32)
                mf16 = jnp.full((16,), tail[lane], jnp.int32).astype(
                    jnp.float32
                )
                j16 = jnp.full((16,), jj, jnp.int32)
                for kk in range(4):
                    v_main = plsc.load_gather(tiles_v.at[slot], [d4[kk], c16])
                    v_tail = plsc.load_gather(tail_v, [d4[kk], c16])
                    val = v_main + (v_tail - v_main) * mf16
                    plsc.store_scatter(rows_v, [j16, d4[kk]], val)
                if lane < 12:
                    off = pl.multiple_of(vt[lane + 4], 128)
                else:
                    off = pl.multiple_of(vtn[lane - 12], 128)
                _fetch(off, slot)
            return carry

        lax.fori_loop(0, _B_PER_W // 16, body, 0)
        for u in range(4):
            _slot_wait(u)
        pltpu.sync_copy(rows_v, out_hbm.at[pl.ds(base, _B_PER_W)])

    return k(indexes, tableT)


def _mm_body(x_ref, wt_ref, b_ref, o_ref):
    o_ref[...] = (
        jnp.dot(x_ref[...], wt_ref[...], preferred_element_type=jnp.float32)
        + b_ref[...]
    )


def _classify_tc(x, W, b):
    blk = 4096
    wt = W.T
    b2 = b.reshape(1, NUM_CLASS)
    return pl.pallas_call(
        _mm_body,
        grid=(BATCH // blk,),
        in_specs=[
            pl.BlockSpec((blk, EMBED_DIM), lambda i: (i, 0)),
            pl.BlockSpec((EMBED_DIM, NUM_CLASS), lambda i: (0, 0)),
            pl.BlockSpec((1, NUM_CLASS), lambda i: (0, 0)),
        ],
        out_specs=pl.BlockSpec((blk, NUM_CLASS), lambda i: (i, 0)),
        out_shape=jax.ShapeDtypeStruct((BATCH, NUM_CLASS), jnp.float32),
    )(x, wt, b2)


def kernel(indexes, table, W, b):
    x = _gather_sc(indexes, table.T)
    return _classify_tc(x, W, b)


# speedup vs baseline: 2.8067x; 1.1555x over previous
import functools

import jax
import jax.numpy as jnp
from jax import lax
from jax.experimental import pallas as pl
from jax.experimental.pallas import tpu as pltpu
from jax.experimental.pallas import tpu_sc as plsc

VOCAB = 1000000
EMBED_DIM = 64
BATCH = 16384
NUM_CLASS = 40

_NC = 2
_NS = 16
_NW = _NC * _NS
_B_PER_W = BATCH // _NW

_TAIL_START = (VOCAB // 128) * 128
_TAIL = VOCAB - _TAIL_START
_VT_MAX = _TAIL_START - 128


def _gather_sc(indexes, tableT):
    mesh = plsc.VectorSubcoreMesh(core_axis_name="c", subcore_axis_name="s")

    @functools.partial(
        pl.kernel,
        mesh=mesh,
        out_type=jax.ShapeDtypeStruct((BATCH, EMBED_DIM), jnp.float32),
        compiler_params=pltpu.CompilerParams(needs_layout_passes=False),
        scratch_types=[
            pltpu.VMEM((_B_PER_W,), jnp.int32),
            pltpu.VMEM((8, EMBED_DIM, 128), jnp.float32),
            pltpu.VMEM((EMBED_DIM, _TAIL), jnp.float32),
            pltpu.VMEM((_B_PER_W // 2, EMBED_DIM), jnp.float32),
            pltpu.SemaphoreType.DMA,
            pltpu.SemaphoreType.DMA,
            pltpu.SemaphoreType.DMA,
            pltpu.SemaphoreType.DMA,
            pltpu.SemaphoreType.DMA,
            pltpu.SemaphoreType.DMA,
            pltpu.SemaphoreType.DMA,
            pltpu.SemaphoreType.DMA,
        ],
    )
    def k(idx_hbm, tableT_hbm, out_hbm, idx_v, tiles_v, tail_v, rows_v,
          s0, s1, s2, s3, s4, s5, s6, s7):
        sems = [s0, s1, s2, s3, s4, s5, s6, s7]
        wid = lax.axis_index("s") * _NC + lax.axis_index("c")
        base = wid * _B_PER_W
        pltpu.sync_copy(idx_hbm.at[pl.ds(base, _B_PER_W)], idx_v)
        pltpu.sync_copy(tableT_hbm.at[:, pl.ds(_TAIL_START, _TAIL)], tail_v)
        d4 = [lax.iota(jnp.int32, 16) + 16 * kk for kk in range(4)]

        def _fetch(off, slot):
            pltpu.async_copy(
                tableT_hbm.at[:, pl.ds(off, 128)], tiles_v.at[slot],
                sems[slot],
            )

        def _slot_wait(slot):
            pltpu.make_async_copy(
                tableT_hbm.at[:, pl.ds(0, 128)], tiles_v.at[slot], sems[slot]
            ).wait()

        iv0 = idx_v[pl.ds(0, 16)]
        vt0 = jnp.minimum(iv0 & ~127, _VT_MAX)
        for u in range(8):
            _fetch(pl.multiple_of(vt0[u], 128), u)

        half_t = _B_PER_W // 32

        def body(t, carry):
            @pl.when(t == half_t)
            def _():
                pltpu.sync_copy(
                    rows_v, out_hbm.at[pl.ds(base, _B_PER_W // 2)]
                )

            iv = idx_v[pl.ds(t * 16, 16)]
            vt = jnp.minimum(iv & ~127, _VT_MAX)
            vin = iv & 127
            tail = (iv >= _TAIL_START).astype(jnp.int32)
            tn = jnp.minimum(t + 1, _B_PER_W // 16 - 1)
            ivn = idx_v[pl.ds(tn * 16, 16)]
            vtn = jnp.minimum(ivn & ~127, _VT_MAX)
            for lane in range(16):
                slot = lane % 8
                _slot_wait(slot)
                jj = t * 16 + lane
                c16 = jnp.full((16,), vin[lane], jnp.int32)
                mf16 = jnp.full((16,), tail[lane], jnp.int32).astype(
                    jnp.float32
                )
                j16 = jnp.full((16,), jj & (_B_PER_W // 2 - 1), jnp.int32)
                for kk in range(4):
                    v_main = plsc.load_gather(tiles_v.at[slot], [d4[kk], c16])
                    v_tail = plsc.load_gather(tail_v, [d4[kk], c16])
                    val = v_main + (v_tail - v_main) * mf16
                    plsc.store_scatter(rows_v, [j16, d4[kk]], val)
                if lane < 8:
                    off = pl.multiple_of(vt[lane + 8], 128)
                else:
                    off = pl.multiple_of(vtn[lane - 8], 128)
                _fetch(off, slot)
            return carry

        lax.fori_loop(0, _B_PER_W // 16, body, 0)
        for u in range(8):
            _slot_wait(u)
        pltpu.sync_copy(
            rows_v, out_hbm.at[pl.ds(base + _B_PER_W // 2, _B_PER_W // 2)]
        )

    return k(indexes, tableT)


def _mm_body(x_ref, wt_ref, b_ref, o_ref):
    o_ref[...] = (
        jnp.dot(x_ref[...], wt_ref[...], preferred_element_type=jnp.float32)
        + b_ref[...]
    )


def _classify_tc(x, W, b):
    blk = 4096
    wt = W.T
    b2 = b.reshape(1, NUM_CLASS)
    return pl.pallas_call(
        _mm_body,
        grid=(BATCH // blk,),
        in_specs=[
            pl.BlockSpec((blk, EMBED_DIM), lambda i: (i, 0)),
            pl.BlockSpec((EMBED_DIM, NUM_CLASS), lambda i: (0, 0)),
            pl.BlockSpec((1, NUM_CLASS), lambda i: (0, 0)),
        ],
        out_specs=pl.BlockSpec((blk, NUM_CLASS), lambda i: (i, 0)),
        out_shape=jax.ShapeDtypeStruct((BATCH, NUM_CLASS), jnp.float32),
    )(x, wt, b2)


def kernel(indexes, table, W, b):
    x = _gather_sc(indexes, table.T)
    return _classify_tc(x, W, b)


# speedup vs baseline: 2.8083x; 1.0005x over previous
import functools

import jax
import jax.numpy as jnp
from jax import lax
from jax.experimental import pallas as pl
from jax.experimental.pallas import tpu as pltpu
from jax.experimental.pallas import tpu_sc as plsc

VOCAB = 1000000
EMBED_DIM = 64
BATCH = 16384
NUM_CLASS = 40

_NC = 2
_NS = 16
_NW = _NC * _NS
_B_PER_W = BATCH // _NW

_TAIL_START = (VOCAB // 128) * 128
_TAIL = VOCAB - _TAIL_START
_VT_MAX = _TAIL_START - 128


def _gather_sc(indexes, tableT):
    mesh = plsc.VectorSubcoreMesh(core_axis_name="c", subcore_axis_name="s")

    @functools.partial(
        pl.kernel,
        mesh=mesh,
        out_type=jax.ShapeDtypeStruct((BATCH, EMBED_DIM), jnp.float32),
        compiler_params=pltpu.CompilerParams(needs_layout_passes=False),
        scratch_types=[
            pltpu.VMEM((_B_PER_W,), jnp.int32),
            pltpu.VMEM((8, EMBED_DIM, 128), jnp.float32),
            pltpu.VMEM((EMBED_DIM, _TAIL), jnp.float32),
            pltpu.VMEM((_B_PER_W // 2, EMBED_DIM), jnp.float32),
            pltpu.SemaphoreType.DMA,
            pltpu.SemaphoreType.DMA,
            pltpu.SemaphoreType.DMA,
            pltpu.SemaphoreType.DMA,
            pltpu.SemaphoreType.DMA,
            pltpu.SemaphoreType.DMA,
            pltpu.SemaphoreType.DMA,
            pltpu.SemaphoreType.DMA,
        ],
    )
    def k(idx_hbm, tableT_hbm, out_hbm, idx_v, tiles_v, tail_v, rows_v,
          s0, s1, s2, s3, s4, s5, s6, s7):
        sems = [s0, s1, s2, s3, s4, s5, s6, s7]
        wid = lax.axis_index("s") * _NC + lax.axis_index("c")
        base = wid * _B_PER_W
        pltpu.sync_copy(idx_hbm.at[pl.ds(base, _B_PER_W)], idx_v)
        pltpu.sync_copy(tableT_hbm.at[:, pl.ds(_TAIL_START, _TAIL)], tail_v)
        d4 = [lax.iota(jnp.int32, 16) + 16 * kk for kk in range(4)]

        def _fetch(off, slot):
            pltpu.async_copy(
                tableT_hbm.at[:, pl.ds(off, 128)], tiles_v.at[slot],
                sems[slot],
            )

        def _slot_wait(slot):
            pltpu.make_async_copy(
                tableT_hbm.at[:, pl.ds(0, 128)], tiles_v.at[slot], sems[slot]
            ).wait()

        iv0 = idx_v[pl.ds(0, 16)]
        vt0 = jnp.minimum(iv0 & ~127, _VT_MAX)
        for u in range(8):
            _fetch(pl.multiple_of(vt0[u], 128), u)

        half_t = _B_PER_W // 32

        def body(t, carry):
            @pl.when(t == half_t)
            def _():
                pltpu.sync_copy(
                    rows_v, out_hbm.at[pl.ds(base, _B_PER_W // 2)]
                )

            iv = idx_v[pl.ds(t * 16, 16)]
            vt = jnp.minimum(iv & ~127, _VT_MAX)
            vin = iv & 127
            tail = (iv >= _TAIL_START).astype(jnp.int32)
            tn = jnp.minimum(t + 1, _B_PER_W // 16 - 1)
            ivn = idx_v[pl.ds(tn * 16, 16)]
            vtn = jnp.minimum(ivn & ~127, _VT_MAX)
            for lane in range(16):
                slot = lane % 8
                _slot_wait(slot)
                jj = t * 16 + lane
                c16 = jnp.full((16,), vin[lane], jnp.int32)
                mf16 = jnp.full((16,), tail[lane], jnp.int32).astype(
                    jnp.float32
                )
                j16 = jnp.full((16,), jj & (_B_PER_W // 2 - 1), jnp.int32)
                for kk in range(4):
                    v_main = plsc.load_gather(tiles_v.at[slot], [d4[kk], c16])
                    v_tail = plsc.load_gather(tail_v, [d4[kk], c16])
                    val = v_main * (1.0 - mf16) + v_tail * mf16
                    plsc.store_scatter(rows_v, [j16, d4[kk]], val)
                if lane < 8:
                    off = pl.multiple_of(vt[lane + 8], 128)
                else:
                    off = pl.multiple_of(vtn[lane - 8], 128)
                _fetch(off, slot)
            return carry

        lax.fori_loop(0, _B_PER_W // 16, body, 0)
        for u in range(8):
            _slot_wait(u)
        pltpu.sync_copy(
            rows_v, out_hbm.at[pl.ds(base + _B_PER_W // 2, _B_PER_W // 2)]
        )

    return k(indexes, tableT)


def _mm_body(x_ref, wt_ref, b_ref, o_ref):
    o_ref[...] = (
        jnp.dot(x_ref[...], wt_ref[...], preferred_element_type=jnp.float32)
        + b_ref[...]
    )


def _classify_tc(x, W, b):
    blk = 4096
    wt = W.T
    b2 = b.reshape(1, NUM_CLASS)
    return pl.pallas_call(
        _mm_body,
        grid=(BATCH // blk,),
        in_specs=[
            pl.BlockSpec((blk, EMBED_DIM), lambda i: (i, 0)),
            pl.BlockSpec((EMBED_DIM, NUM_CLASS), lambda i: (0, 0)),
            pl.BlockSpec((1, NUM_CLASS), lambda i: (0, 0)),
        ],
        out_specs=pl.BlockSpec((blk, NUM_CLASS), lambda i: (i, 0)),
        out_shape=jax.ShapeDtypeStruct((BATCH, NUM_CLASS), jnp.float32),
    )(x, wt, b2)


def kernel(indexes, table, W, b):
    x = _gather_sc(indexes, table.T)
    return _classify_tc(x, W, b)
